# baseline (speedup 1.0000x reference)
import dataclasses
import functools

import jax
import jax.numpy as jnp
from jax.experimental import pallas as pl
from jax.experimental.pallas import tpu as pltpu
from jax.experimental.pallas import tpu_sc as plsc

F32 = jnp.float32
BF16 = jnp.bfloat16
LANES = 16


def _ln(x, scale, bias, eps=1e-6):
    mu = jnp.mean(x, axis=-1, keepdims=True)
    var = jnp.mean((x - mu) ** 2, axis=-1, keepdims=True)
    return (x - mu) / jnp.sqrt(var + eps) * scale + bias


def _stage_a_kernel(x_ref, s_ref, b_ref, normed_ref, pooled_ref):
    xb = x_ref[0]
    normed = _ln(xb, s_ref[...], b_ref[...])
    normed_ref[0] = normed
    pooled_ref[0] = jnp.mean(normed, axis=0, keepdims=True)


def _stage_a2_kernel(pooled_ref, rw_ref, rb_ref, logits_ref):
    B, E = pooled_ref.shape[0], rw_ref.shape[1]
    logits = jnp.dot(pooled_ref[...].astype(BF16), rw_ref[...].astype(BF16),
                     preferred_element_type=F32) + rb_ref[...]
    pad = jnp.full((B, LANES - E), -1e30, F32)
    logits_ref[...] = jnp.concatenate([logits, pad], axis=1)


def _sc_route(logits16, B):
    mesh = plsc.VectorSubcoreMesh(core_axis_name="c", subcore_axis_name="s")
    cp = pltpu.CompilerParams()
    if "needs_layout_passes" in pltpu.CompilerParams.__dataclass_fields__:
        cp = dataclasses.replace(cp, needs_layout_passes=False)

    @pl.kernel(
        compiler_params=cp,
        out_type=[
            jax.ShapeDtypeStruct((B, LANES), F32),
            jax.ShapeDtypeStruct((B, LANES), F32),
        ],
        mesh=mesh,
        scratch_types=[
            pltpu.VMEM((LANES,), F32),
            pltpu.VMEM((LANES,), F32),
            pltpu.VMEM((LANES,), F32),
        ],
    )
    def route_kernel(l_hbm, g_hbm, p_hbm, lrow, grow, prow):
        c = jax.lax.axis_index("c")
        s = jax.lax.axis_index("s")

        @pl.when(jnp.logical_and(c == 0, s < B))
        def _():
            pltpu.sync_copy(l_hbm.at[s], lrow)
            v = lrow[...]
            m = jnp.max(v)
            ex = jnp.exp(v - m)
            p = ex / jnp.full((LANES,), jnp.sum(ex), F32)
            iota = jax.lax.iota(jnp.int32, LANES)
            v1 = jnp.max(p)
            i1 = jnp.min(jnp.where(p == v1, iota, LANES))
            masked = jnp.where(iota == i1, -1.0, p)
            v2 = jnp.max(masked)
            i2 = jnp.min(jnp.where(masked == v2, iota, LANES))
            top = (jnp.where(iota == i1, v1, 0.0)
                   + jnp.where(iota == i2, v2, 0.0))
            gates = top / jnp.full((LANES,), v1 + v2, F32)
            grow[...] = gates
            prow[...] = p
            pltpu.sync_copy(grow, g_hbm.at[s])
            pltpu.sync_copy(prow, p_hbm.at[s])

    return route_kernel(logits16)


def _stage_b_kernel(g_ref, w_ref, x_ref, g16_ref, p16_ref,
                    out_ref, aux_ref, *, H, dh, B, E):
    e = pl.program_id(0)

    @pl.when(e == 0)
    def _():
        out_ref[...] = jnp.zeros_like(out_ref)
        cnt = (g16_ref[...] > 0.0).astype(F32)
        frac = jnp.sum(cnt, axis=0, keepdims=True) / (B * 2)
        mean_p = jnp.mean(p16_ref[...], axis=0, keepdims=True)
        aux_ref[...] = E * jnp.sum(frac * mean_p, keepdims=True)

    for b in range(B):
        g = g_ref[b * E + e]

        @pl.when(g > 0.0)
        def _():
            xb = x_ref[b]
            pieces = []
            for h in range(H):
                w = w_ref[0, h].astype(BF16)
                xs = xb[:, h * dh:(h + 1) * dh].astype(BF16)
                pieces.append(jnp.dot(w, xs, preferred_element_type=F32))
            out_ref[b] += jnp.concatenate(pieces, axis=1) * g


def _stage_c_kernel(x_ref, m_ref, pw_ref, pb_ref, s2_ref, b2_ref,
                    w1_ref, b1_ref, w2_ref, b2b_ref, out_ref, *, hid_chunk):
    u = x_ref[...] + jnp.dot(m_ref[...].astype(BF16), pw_ref[...].astype(BF16),
                             preferred_element_type=F32) + pb_ref[...]
    n2 = _ln(u, s2_ref[...], b2_ref[...]).astype(BF16)
    hid = w1_ref.shape[1]
    acc = u + b2b_ref[...]
    for j in range(0, hid, hid_chunk):
        h1 = jnp.dot(n2, w1_ref[:, j:j + hid_chunk].astype(BF16),
                     preferred_element_type=F32) + b1_ref[:, j:j + hid_chunk]
        h1 = (0.5 * h1 * (1.0 + jax.lax.erf(h1 * 0.7071067811865476))).astype(BF16)
        acc = acc + jnp.dot(h1, w2_ref[j:j + hid_chunk, :].astype(BF16),
                            preferred_element_type=F32)
    out_ref[...] = acc


def kernel(x, ln1_scale, ln1_bias, router_w, router_b, expert_w, proj_w,
           proj_b, ln2_scale, ln2_bias, mlp_w1, mlp_b1, mlp_w2, mlp_b2):
    B, N, D = x.shape
    E, H = expert_w.shape[0], expert_w.shape[1]
    dh = D // H
    hid = mlp_w1.shape[1]

    normed, pooled = pl.pallas_call(
        _stage_a_kernel,
        grid=(B,),
        in_specs=[
            pl.BlockSpec((1, N, D), lambda b: (b, 0, 0)),
            pl.BlockSpec((1, D), lambda b: (0, 0)),
            pl.BlockSpec((1, D), lambda b: (0, 0)),
        ],
        out_specs=[
            pl.BlockSpec((1, N, D), lambda b: (b, 0, 0)),
            pl.BlockSpec((1, 1, D), lambda b: (b, 0, 0)),
        ],
        out_shape=[
            jax.ShapeDtypeStruct((B, N, D), F32),
            jax.ShapeDtypeStruct((B, 1, D), F32),
        ],
    )(x, ln1_scale.reshape(1, D), ln1_bias.reshape(1, D))
    pooled = pooled.reshape(B, D)

    logits16 = pl.pallas_call(
        _stage_a2_kernel,
        out_shape=jax.ShapeDtypeStruct((B, LANES), F32),
    )(pooled, router_w, router_b.reshape(1, E))

    gmat16, probs16 = _sc_route(logits16, B)
    gflat = gmat16[:, :E].reshape(B * E)

    mixed, aux = pl.pallas_call(
        functools.partial(_stage_b_kernel, H=H, dh=dh, B=B, E=E),
        grid_spec=pltpu.PrefetchScalarGridSpec(
            num_scalar_prefetch=1,
            grid=(E,),
            in_specs=[
                pl.BlockSpec((1, H, N, N), lambda e, g: (e, 0, 0, 0)),
                pl.BlockSpec((B, N, D), lambda e, g: (0, 0, 0)),
                pl.BlockSpec((B, LANES), lambda e, g: (0, 0)),
                pl.BlockSpec((B, LANES), lambda e, g: (0, 0)),
            ],
            out_specs=[
                pl.BlockSpec((B, N, D), lambda e, g: (0, 0, 0)),
                pl.BlockSpec((1, 1), lambda e, g: (0, 0)),
            ],
        ),
        out_shape=[
            jax.ShapeDtypeStruct((B, N, D), F32),
            jax.ShapeDtypeStruct((1, 1), F32),
        ],
    )(gflat, expert_w, normed, gmat16, probs16)

    R = 512
    rows = B * N
    y = pl.pallas_call(
        functools.partial(_stage_c_kernel, hid_chunk=768),
        grid=(rows // R,),
        in_specs=[
            pl.BlockSpec((R, D), lambda r: (r, 0)),
            pl.BlockSpec((R, D), lambda r: (r, 0)),
            pl.BlockSpec((D, D), lambda r: (0, 0)),
            pl.BlockSpec((1, D), lambda r: (0, 0)),
            pl.BlockSpec((1, D), lambda r: (0, 0)),
            pl.BlockSpec((1, D), lambda r: (0, 0)),
            pl.BlockSpec((D, hid), lambda r: (0, 0)),
            pl.BlockSpec((1, hid), lambda r: (0, 0)),
            pl.BlockSpec((hid, D), lambda r: (0, 0)),
            pl.BlockSpec((1, D), lambda r: (0, 0)),
        ],
        out_specs=pl.BlockSpec((R, D), lambda r: (r, 0)),
        out_shape=jax.ShapeDtypeStruct((rows, D), F32),
    )(x.reshape(rows, D), mixed.reshape(rows, D), proj_w,
      proj_b.reshape(1, D), ln2_scale.reshape(1, D), ln2_bias.reshape(1, D),
      mlp_w1, mlp_b1.reshape(1, hid), mlp_w2, mlp_b2.reshape(1, D))

    return (y.reshape(B, N, D), aux.reshape(()))
